# baseline (speedup 1.0000x reference)
import math

import jax
import jax.numpy as jnp
import numpy as np
from jax import lax
from jax.experimental import pallas as pl
from jax.experimental.pallas import tpu as pltpu
from jax.experimental.pallas import tpu_sc as plsc
from jax._src import config as _jax_config

L = 16
MIN_RES = 16
MAX_RES = 4096
LOG2_T = 19
F = 2
T = 2 ** LOG2_T
N_POINTS = 524288

_GROWTH = math.exp((math.log(MAX_RES) - math.log(MIN_RES)) / (L - 1))
_LEVEL_RES = np.floor(MIN_RES * (_GROWTH ** np.arange(L, dtype=np.float64))).astype(np.int32)
_P1 = np.uint32(2654435761).astype(np.int32)
_P2 = np.int32(805459861)
_MASK = np.int32(T - 1)
_MASK_LO = np.int32(127)

NW = 32
PW = N_POINTS // NW
C = 128
ITERS = PW // C
GROUPS = C // 16

DG = 2
SL = L - DG
NELEM = C * SL * 8
NIDX = NELEM // 128

_R0 = int(_LEVEL_RES[0])
_R1 = int(_LEVEL_RES[1])
_R2 = int(_LEVEL_RES[2])
_N0 = (_R0 + 1) ** 3
_N1 = (_R1 + 1) ** 3
_N2 = (_R2 + 1) ** 3
_NP0 = (_N0 + 127) // 128 * 128
_NP1 = (_N1 + 127) // 128 * 128
_NP2 = (_N2 + 127) // 128 * 128

TROWS_W = T * L // NW
SLAB = 2048
PITERS = TROWS_W // SLAB


def _pack_body(src, packed, stage, pstage):
    i32 = jnp.int32
    wid = lax.axis_index("s") * i32(2) + lax.axis_index("c")

    @pl.loop(0, PITERS)
    def _slab(it):
        rbase = wid * i32(TROWS_W) + it * i32(SLAB)
        pltpu.sync_copy(src.at[pl.ds(rbase * i32(2), 2 * SLAB)], stage)
        for b in range(16):
            for i in range(8):
                a = stage[pl.ds(b * 256 + i * 16, 16)]
                bb = stage[pl.ds(b * 256 + 128 + i * 16, 16)]
                p = plsc.bitcast(
                    plsc.pack(a, bb, format=plsc.PackFormat.INTERLEAVED),
                    jnp.int32)
                pstage[pl.ds(b * 128 + i * 16, 16)] = p
        pltpu.sync_copy(pstage, packed.at[pl.ds(rbase, SLAB)])


def _unpack16(v):
    fa, fb = plsc.unpack(plsc.bitcast(v, jnp.bfloat16),
                         format=plsc.PackFormat.INTERLEAVED)
    return fa.astype(jnp.float32), fb.astype(jnp.float32)


def _body(xs, ys, zs, packed, out_hbm,
          xsA, ysA, zsA, xsB, ysB, zsB,
          idxA, idxB, rowsA, rowsB, grid0, grid1, out_v,
          semA, semB, csem):
    i32 = jnp.int32
    wid = lax.axis_index("s") * i32(2) + lax.axis_index("c")

    lane = jnp.arange(16, dtype=jnp.int32)
    pat = lane * i32(32)

    def build_grid(res, n, npad, grid, lvl):
        rp = res + 1
        off = i32(lvl * T)
        piece = min(npad, NIDX * 128)
        for p0 in range(0, npad, piece):
            pn = min(piece, npad - p0)

            @pl.loop(0, pn, step=16)
            def _mk(s):
                d = jnp.minimum(i32(p0) + s + lane, i32(n - 1))
                ix = lax.div(d, i32(rp * rp))
                rem = d - ix * i32(rp * rp)
                iy = lax.div(rem, i32(rp))
                iz = rem - iy * i32(rp)
                h = ix ^ (iy * _P1) ^ (iz * _P2)
                e = (h & _MASK) + off
                idxA[s >> i32(7), pl.ds(s & i32(127), 16)] = e

            @pl.loop(0, pn // 128)
            def _fire(j):
                pltpu.async_copy(packed.at[idxA.at[j]],
                                 grid.at[pl.ds(i32(p0) + j * i32(128), 128)],
                                 semA)

            pltpu.make_async_copy(packed.at[pl.ds(0, pn)],
                                  grid.at[pl.ds(p0, pn)], semA).wait()

    def load_coords(chunk, xv, yv, zv):
        pbase = wid * i32(PW) + chunk * i32(C)
        da = pltpu.async_copy(xs.at[pl.ds(pbase, C)], xv, csem)
        db = pltpu.async_copy(ys.at[pl.ds(pbase, C)], yv, csem)
        dc = pltpu.async_copy(zs.at[pl.ds(pbase, C)], zv, csem)
        da.wait()
        db.wait()
        dc.wait()

    def index_pass(xv, yv, zv, idx_v):
        @pl.loop(0, GROUPS)
        def _idx(g):
            gb = g * i32(16)
            x16 = xv[pl.ds(gb, 16)]
            y16 = yv[pl.ds(gb, 16)]
            z16 = zv[pl.ds(gb, 16)]
            for l in range(DG, L):
                r = jnp.float32(_LEVEL_RES[l])
                ix = (x16 * r).astype(jnp.int32)
                iy = (y16 * r).astype(jnp.int32)
                iz = (z16 * r).astype(jnp.int32)
                hx0 = ix
                hx1 = ix + i32(1)
                hy0 = iy * _P1
                hy1 = hy0 + _P1
                hz0 = iz * _P2
                hz1 = hz0 + _P2
                off = i32(l * T)
                for cx in range(2):
                    hx = hx1 if cx else hx0
                    for cy in range(2):
                        hxy = hx ^ (hy1 if cy else hy0)
                        for cz in range(2):
                            h = hxy ^ (hz1 if cz else hz0)
                            e = (h & _MASK) + off
                            blk = (l - DG) * 8 + cx * 4 + cy * 2 + cz
                            idx_v[blk, pl.ds(gb, 16)] = e

    def fire(idx_v, rows_v, sem):
        @pl.loop(0, NIDX, unroll=4)
        def _gather(j):
            pltpu.async_copy(packed.at[idx_v.at[j]],
                             rows_v.at[pl.ds(j * i32(128), 128)], sem)

    def drain(rows_v, sem):
        pltpu.make_async_copy(packed.at[pl.ds(0, NELEM)], rows_v, sem).wait()

    def combine(chunk, xv, yv, zv, rows_v):
        pbase = wid * i32(PW) + chunk * i32(C)

        @pl.loop(0, GROUPS)
        def _combine(g):
            gb = g * i32(16)
            x16 = xv[pl.ds(gb, 16)]
            y16 = yv[pl.ds(gb, 16)]
            z16 = zv[pl.ds(gb, 16)]
            for l in range(L):
                r = jnp.float32(_LEVEL_RES[l])
                sx = x16 * r
                sy = y16 * r
                sz = z16 * r
                ix = sx.astype(jnp.int32)
                iy = sy.astype(jnp.int32)
                iz = sz.astype(jnp.int32)
                wx = sx - ix.astype(jnp.float32)
                wy = sy - iy.astype(jnp.float32)
                wz = sz - iz.astype(jnp.float32)
                f0 = []
                f1 = []
                if l < DG:
                    rp = int(_LEVEL_RES[l]) + 1
                    grid = (grid0, grid1)[l]
                    d000 = ix * i32(rp * rp) + iy * i32(rp) + iz
                    for cx in range(2):
                        for cy in range(2):
                            for cz in range(2):
                                d = d000 + i32(cx * rp * rp + cy * rp + cz)
                                fa, fb = _unpack16(plsc.load_gather(grid, [d]))
                                f0.append(fa)
                                f1.append(fb)
                else:
                    base = i32((l - DG) * 8 * C) + gb
                    for c in range(8):
                        fa, fb = _unpack16(rows_v[pl.ds(base + i32(c * C), 16)])
                        f0.append(fa)
                        f1.append(fb)
                res = []
                for f in (f0, f1):
                    c00 = f[0] + wz * (f[1] - f[0])
                    c01 = f[2] + wz * (f[3] - f[2])
                    c10 = f[4] + wz * (f[5] - f[4])
                    c11 = f[6] + wz * (f[7] - f[6])
                    c0 = c00 + wy * (c01 - c00)
                    c1 = c10 + wy * (c11 - c10)
                    res.append(c0 + wx * (c1 - c0))
                dst = pat + gb * i32(32) + i32(l * 2)
                plsc.store_scatter(out_v, [dst], res[0])
                plsc.store_scatter(out_v, [dst + i32(1)], res[1])

        pltpu.sync_copy(out_v, out_hbm.at[pl.ds(pbase * i32(32), C * 32)])

    build_grid(_R0, _N0, _NP0, grid0, 0)
    build_grid(_R1, _N1, _NP1, grid1, 1)

    load_coords(i32(0), xsA, ysA, zsA)
    index_pass(xsA, ysA, zsA, idxA)
    fire(idxA, rowsA, semA)

    @pl.loop(0, ITERS // 2)
    def _pair(k):
        even = k * i32(2)
        odd = even + i32(1)
        nxt = (even + i32(2)) & i32(ITERS - 1)

        load_coords(odd, xsB, ysB, zsB)
        index_pass(xsB, ysB, zsB, idxB)
        fire(idxB, rowsB, semB)

        drain(rowsA, semA)
        combine(even, xsA, ysA, zsA, rowsA)

        load_coords(nxt, xsA, ysA, zsA)
        index_pass(xsA, ysA, zsA, idxA)
        fire(idxA, rowsA, semA)

        drain(rowsB, semB)
        combine(odd, xsB, ysB, zsB, rowsB)

    drain(rowsA, semA)


def kernel(x, hash_table):
    with _jax_config.enable_x64(False):
        return _run(x, hash_table)


def _run(x, hash_table):
    xt = x.T
    xs, ys, zs = xt[0], xt[1], xt[2]
    native = hash_table.reshape(T * L // 128, 128, F).transpose(0, 2, 1).reshape(-1)

    mesh = plsc.VectorSubcoreMesh(core_axis_name="c", subcore_axis_name="s")

    packed = pl.kernel(
        _pack_body,
        out_type=jax.ShapeDtypeStruct((T * L,), jnp.int32),
        mesh=mesh,
        compiler_params=pltpu.CompilerParams(needs_layout_passes=False),
        scratch_types=[
            pltpu.VMEM((2 * SLAB,), jnp.float32),
            pltpu.VMEM((SLAB,), jnp.int32),
        ],
    )(native)

    out = pl.kernel(
        _body,
        out_type=jax.ShapeDtypeStruct((N_POINTS * L * F,), jnp.float32),
        mesh=mesh,
        compiler_params=pltpu.CompilerParams(needs_layout_passes=False),
        scratch_types=[
            pltpu.VMEM((C,), jnp.float32),
            pltpu.VMEM((C,), jnp.float32),
            pltpu.VMEM((C,), jnp.float32),
            pltpu.VMEM((C,), jnp.float32),
            pltpu.VMEM((C,), jnp.float32),
            pltpu.VMEM((C,), jnp.float32),
            pltpu.VMEM((NIDX, 128), jnp.int32),
            pltpu.VMEM((NIDX, 128), jnp.int32),
            pltpu.VMEM((NELEM,), jnp.int32),
            pltpu.VMEM((NELEM,), jnp.int32),
            pltpu.VMEM((_NP0,), jnp.int32),
            pltpu.VMEM((_NP1,), jnp.int32),
            pltpu.VMEM((C * 32,), jnp.float32),
            pltpu.SemaphoreType.DMA,
            pltpu.SemaphoreType.DMA,
            pltpu.SemaphoreType.DMA,
        ],
    )(xs, ys, zs, packed)
    return out.reshape(N_POINTS, L * F)


# speedup vs baseline: 1.1226x; 1.1226x over previous
import math

import jax
import jax.numpy as jnp
import numpy as np
from jax import lax
from jax.experimental import pallas as pl
from jax.experimental.pallas import tpu as pltpu
from jax.experimental.pallas import tpu_sc as plsc
from jax._src import config as _jax_config

L = 16
MIN_RES = 16
MAX_RES = 4096
LOG2_T = 19
F = 2
T = 2 ** LOG2_T
N_POINTS = 524288

_GROWTH = math.exp((math.log(MAX_RES) - math.log(MIN_RES)) / (L - 1))
_LEVEL_RES = np.floor(MIN_RES * (_GROWTH ** np.arange(L, dtype=np.float64))).astype(np.int32)
_P1 = np.uint32(2654435761).astype(np.int32)
_P2 = np.int32(805459861)
_MASK = np.int32(T - 1)
_MASK_LO = np.int32(127)

NW = 32
PW = N_POINTS // NW
C = 64
ITERS = PW // C
GROUPS = C // 16

DG = 3
SL = L - DG
NELEM = C * SL * 8
NIDX = NELEM // 128

_R0 = int(_LEVEL_RES[0])
_R1 = int(_LEVEL_RES[1])
_R2 = int(_LEVEL_RES[2])
_N0 = (_R0 + 1) ** 3
_N1 = (_R1 + 1) ** 3
_N2 = (_R2 + 1) ** 3
_NP0 = (_N0 + 127) // 128 * 128
_NP1 = (_N1 + 127) // 128 * 128
_NP2 = (_N2 + 127) // 128 * 128

TROWS_W = T * L // NW
SLAB = 2048
PITERS = TROWS_W // SLAB


def _pack_body(src, packed, stage, pstage):
    i32 = jnp.int32
    wid = lax.axis_index("s") * i32(2) + lax.axis_index("c")

    @pl.loop(0, PITERS)
    def _slab(it):
        rbase = wid * i32(TROWS_W) + it * i32(SLAB)
        pltpu.sync_copy(src.at[pl.ds(rbase * i32(2), 2 * SLAB)], stage)
        for b in range(16):
            for i in range(8):
                a = stage[pl.ds(b * 256 + i * 16, 16)]
                bb = stage[pl.ds(b * 256 + 128 + i * 16, 16)]
                p = plsc.bitcast(
                    plsc.pack(a, bb, format=plsc.PackFormat.INTERLEAVED),
                    jnp.int32)
                pstage[pl.ds(b * 128 + i * 16, 16)] = p
        pltpu.sync_copy(pstage, packed.at[pl.ds(rbase, SLAB)])


def _unpack16(v):
    fa, fb = plsc.unpack(plsc.bitcast(v, jnp.bfloat16),
                         format=plsc.PackFormat.INTERLEAVED)
    return fa.astype(jnp.float32), fb.astype(jnp.float32)


def _body(xs, ys, zs, packed, out_hbm,
          xsA, ysA, zsA, xsB, ysB, zsB,
          idxA, idxB, rowsA, rowsB, grid0, grid1, grid2, out_v,
          semA, semB, csem):
    i32 = jnp.int32
    wid = lax.axis_index("s") * i32(2) + lax.axis_index("c")

    lane = jnp.arange(16, dtype=jnp.int32)
    pat = lane * i32(32)

    def build_grid(res, n, npad, grid, lvl):
        rp = res + 1
        off = i32(lvl * T)
        piece = min(npad, NIDX * 128)
        for p0 in range(0, npad, piece):
            pn = min(piece, npad - p0)

            @pl.loop(0, pn, step=16)
            def _mk(s):
                d = jnp.minimum(i32(p0) + s + lane, i32(n - 1))
                ix = lax.div(d, i32(rp * rp))
                rem = d - ix * i32(rp * rp)
                iy = lax.div(rem, i32(rp))
                iz = rem - iy * i32(rp)
                h = ix ^ (iy * _P1) ^ (iz * _P2)
                e = (h & _MASK) + off
                idxA[s >> i32(7), pl.ds(s & i32(127), 16)] = e

            @pl.loop(0, pn // 128)
            def _fire(j):
                pltpu.async_copy(packed.at[idxA.at[j]],
                                 grid.at[pl.ds(i32(p0) + j * i32(128), 128)],
                                 semA)

            pltpu.make_async_copy(packed.at[pl.ds(0, pn)],
                                  grid.at[pl.ds(p0, pn)], semA).wait()

    def load_coords(chunk, xv, yv, zv):
        pbase = wid * i32(PW) + chunk * i32(C)
        da = pltpu.async_copy(xs.at[pl.ds(pbase, C)], xv, csem)
        db = pltpu.async_copy(ys.at[pl.ds(pbase, C)], yv, csem)
        dc = pltpu.async_copy(zs.at[pl.ds(pbase, C)], zv, csem)
        da.wait()
        db.wait()
        dc.wait()

    def index_pass(xv, yv, zv, idx_v):
        @pl.loop(0, GROUPS)
        def _idx(g):
            gb = g * i32(16)
            x16 = xv[pl.ds(gb, 16)]
            y16 = yv[pl.ds(gb, 16)]
            z16 = zv[pl.ds(gb, 16)]
            for l in range(DG, L):
                r = jnp.float32(_LEVEL_RES[l])
                ix = (x16 * r).astype(jnp.int32)
                iy = (y16 * r).astype(jnp.int32)
                iz = (z16 * r).astype(jnp.int32)
                hx0 = ix
                hx1 = ix + i32(1)
                hy0 = iy * _P1
                hy1 = hy0 + _P1
                hz0 = iz * _P2
                hz1 = hz0 + _P2
                off = i32(l * T)
                for cx in range(2):
                    hx = hx1 if cx else hx0
                    for cy in range(2):
                        hxy = hx ^ (hy1 if cy else hy0)
                        for cz in range(2):
                            h = hxy ^ (hz1 if cz else hz0)
                            e = (h & _MASK) + off
                            blk = (l - DG) * 8 + cx * 4 + cy * 2 + cz
                            col = i32((blk & 1) * 64) + gb
                            idx_v[blk >> 1, pl.ds(col, 16)] = e

    def fire(idx_v, rows_v, sem):
        @pl.loop(0, NIDX, unroll=4)
        def _gather(j):
            pltpu.async_copy(packed.at[idx_v.at[j]],
                             rows_v.at[pl.ds(j * i32(128), 128)], sem)

    def drain(rows_v, sem):
        pltpu.make_async_copy(packed.at[pl.ds(0, NELEM)], rows_v, sem).wait()

    def combine(chunk, xv, yv, zv, rows_v):
        pbase = wid * i32(PW) + chunk * i32(C)

        @pl.loop(0, GROUPS)
        def _combine(g):
            gb = g * i32(16)
            x16 = xv[pl.ds(gb, 16)]
            y16 = yv[pl.ds(gb, 16)]
            z16 = zv[pl.ds(gb, 16)]
            for l in range(L):
                r = jnp.float32(_LEVEL_RES[l])
                sx = x16 * r
                sy = y16 * r
                sz = z16 * r
                ix = sx.astype(jnp.int32)
                iy = sy.astype(jnp.int32)
                iz = sz.astype(jnp.int32)
                wx = sx - ix.astype(jnp.float32)
                wy = sy - iy.astype(jnp.float32)
                wz = sz - iz.astype(jnp.float32)
                f0 = []
                f1 = []
                if l < DG:
                    rp = int(_LEVEL_RES[l]) + 1
                    grid = (grid0, grid1, grid2)[l]
                    d000 = ix * i32(rp * rp) + iy * i32(rp) + iz
                    for cx in range(2):
                        for cy in range(2):
                            for cz in range(2):
                                d = d000 + i32(cx * rp * rp + cy * rp + cz)
                                fa, fb = _unpack16(plsc.load_gather(grid, [d]))
                                f0.append(fa)
                                f1.append(fb)
                else:
                    base = i32((l - DG) * 8 * C) + gb
                    for c in range(8):
                        fa, fb = _unpack16(rows_v[pl.ds(base + i32(c * C), 16)])
                        f0.append(fa)
                        f1.append(fb)
                res = []
                for f in (f0, f1):
                    c00 = f[0] + wz * (f[1] - f[0])
                    c01 = f[2] + wz * (f[3] - f[2])
                    c10 = f[4] + wz * (f[5] - f[4])
                    c11 = f[6] + wz * (f[7] - f[6])
                    c0 = c00 + wy * (c01 - c00)
                    c1 = c10 + wy * (c11 - c10)
                    res.append(c0 + wx * (c1 - c0))
                dst = pat + gb * i32(32) + i32(l * 2)
                plsc.store_scatter(out_v, [dst], res[0])
                plsc.store_scatter(out_v, [dst + i32(1)], res[1])

        pltpu.sync_copy(out_v, out_hbm.at[pl.ds(pbase * i32(32), C * 32)])

    build_grid(_R0, _N0, _NP0, grid0, 0)
    build_grid(_R1, _N1, _NP1, grid1, 1)
    build_grid(_R2, _N2, _NP2, grid2, 2)

    load_coords(i32(0), xsA, ysA, zsA)
    index_pass(xsA, ysA, zsA, idxA)
    fire(idxA, rowsA, semA)

    @pl.loop(0, ITERS // 2)
    def _pair(k):
        even = k * i32(2)
        odd = even + i32(1)
        nxt = (even + i32(2)) & i32(ITERS - 1)

        load_coords(odd, xsB, ysB, zsB)
        index_pass(xsB, ysB, zsB, idxB)
        fire(idxB, rowsB, semB)

        drain(rowsA, semA)
        combine(even, xsA, ysA, zsA, rowsA)

        load_coords(nxt, xsA, ysA, zsA)
        index_pass(xsA, ysA, zsA, idxA)
        fire(idxA, rowsA, semA)

        drain(rowsB, semB)
        combine(odd, xsB, ysB, zsB, rowsB)

    drain(rowsA, semA)


def kernel(x, hash_table):
    with _jax_config.enable_x64(False):
        return _run(x, hash_table)


def _run(x, hash_table):
    xt = x.T
    xs, ys, zs = xt[0], xt[1], xt[2]
    native = hash_table.reshape(T * L // 128, 128, F).transpose(0, 2, 1).reshape(-1)

    mesh = plsc.VectorSubcoreMesh(core_axis_name="c", subcore_axis_name="s")

    packed = pl.kernel(
        _pack_body,
        out_type=jax.ShapeDtypeStruct((T * L,), jnp.int32),
        mesh=mesh,
        compiler_params=pltpu.CompilerParams(needs_layout_passes=False),
        scratch_types=[
            pltpu.VMEM((2 * SLAB,), jnp.float32),
            pltpu.VMEM((SLAB,), jnp.int32),
        ],
    )(native)

    out = pl.kernel(
        _body,
        out_type=jax.ShapeDtypeStruct((N_POINTS * L * F,), jnp.float32),
        mesh=mesh,
        compiler_params=pltpu.CompilerParams(needs_layout_passes=False),
        scratch_types=[
            pltpu.VMEM((C,), jnp.float32),
            pltpu.VMEM((C,), jnp.float32),
            pltpu.VMEM((C,), jnp.float32),
            pltpu.VMEM((C,), jnp.float32),
            pltpu.VMEM((C,), jnp.float32),
            pltpu.VMEM((C,), jnp.float32),
            pltpu.VMEM((NIDX, 128), jnp.int32),
            pltpu.VMEM((NIDX, 128), jnp.int32),
            pltpu.VMEM((NELEM,), jnp.int32),
            pltpu.VMEM((NELEM,), jnp.int32),
            pltpu.VMEM((_NP0,), jnp.int32),
            pltpu.VMEM((_NP1,), jnp.int32),
            pltpu.VMEM((_NP2,), jnp.int32),
            pltpu.VMEM((C * 32,), jnp.float32),
            pltpu.SemaphoreType.DMA,
            pltpu.SemaphoreType.DMA,
            pltpu.SemaphoreType.DMA,
        ],
    )(xs, ys, zs, packed)
    return out.reshape(N_POINTS, L * F)
